# Initial kernel scaffold; baseline (speedup 1.0000x reference)
import jax
import jax.numpy as jnp
from jax.experimental import pallas as pl


def kernel(x, kernel):
    raise NotImplementedError("write your pallas kernel here")


# speedup vs baseline: 28.6265x; 28.6265x over previous
import numpy as np
import jax
import jax.numpy as jnp
from jax.experimental import pallas as pl

_B, _D, _K = 512, 256, 1024
_SIDE = 32
_SCALE = float(1.0 / (2.0 * np.sqrt(2.0 * np.pi)))


def _som_body(x_ref, w_ref, out_ref):
    x = x_ref[...]
    w = w_ref[...]
    xw = jax.lax.dot_general(
        x, w, (((1,), (0,)), ((), ())),
        preferred_element_type=jnp.float32,
        precision=jax.lax.Precision.HIGHEST,
    )
    x2 = jnp.sum(x * x, axis=1, keepdims=True)
    w2 = jnp.sum(w * w, axis=0, keepdims=True)
    norms2 = (x2 + w2) - 2.0 * xw

    kidx = jax.lax.broadcasted_iota(jnp.int32, (_B, _K), 1)
    minv = jnp.min(norms2, axis=1, keepdims=True)
    wta = jnp.min(jnp.where(norms2 <= minv, kidx, _K), axis=1, keepdims=True)

    wr = (wta // _SIDE).astype(jnp.float32)
    wc = (wta % _SIDE).astype(jnp.float32)
    kr = (kidx // _SIDE).astype(jnp.float32)
    kc = (kidx % _SIDE).astype(jnp.float32)
    d2 = (kr - wr) ** 2 + (kc - wc) ** 2
    out_ref[...] = norms2 * (jnp.exp(-0.125 * d2) * _SCALE)


def kernel(x, kernel):
    return pl.pallas_call(
        _som_body,
        out_shape=jax.ShapeDtypeStruct((_B, _K), jnp.float32),
    )(x, kernel)
